# baseline (speedup 1.0000x reference)
import functools

import jax
import jax.numpy as jnp
from jax import lax
from jax.experimental import pallas as pl
from jax.experimental.pallas import tpu as pltpu
from jax.experimental.pallas import tpu_sc as plsc

try:
    _INFO = plsc.get_sparse_core_info()
    _NC = _INFO.num_cores
    _NS = _INFO.num_subcores
except ValueError:
    _NC, _NS = 2, 16
_NW = _NC * _NS
_IDXW = 128
_BLK = 2048
_SUB = 256


@functools.partial(jax.jit, static_argnums=(2, 3))
def _sc_gather(table, x, batch, dim):
    b_per_w = batch // _NW
    chunks = b_per_w // _IDXW
    pack = 128 // dim
    mesh = plsc.VectorSubcoreMesh(core_axis_name="c", subcore_axis_name="s")

    @functools.partial(
        pl.kernel,
        mesh=mesh,
        compiler_params=pltpu.CompilerParams(
            use_tc_tiling_on_sc=False, needs_layout_passes=False
        ),
        out_type=jax.ShapeDtypeStruct((2, batch // pack, 128), jnp.float32),
        scratch_types=[
            pltpu.VMEM((b_per_w, 2), jnp.int32),
            pltpu.VMEM((b_per_w,), jnp.int32),
            pltpu.VMEM((b_per_w,), jnp.int32),
            pltpu.VMEM((b_per_w, dim), jnp.float32),
            pltpu.VMEM((b_per_w, dim), jnp.float32),
            pltpu.SemaphoreType.DMA,
        ],
    )
    def gather_k(table_hbm, x_hbm, out_hbm, idx_v, idxw_v, idxl_v,
                 roww_v, rowl_v, sem):
        wid = lax.axis_index("s") * _NC + lax.axis_index("c")
        base = wid * b_per_w
        pltpu.sync_copy(x_hbm.at[pl.ds(base, b_per_w)], idx_v)
        lanes = lax.iota(jnp.int32, 16)
        col0 = jnp.zeros((16,), jnp.int32)
        col1 = jnp.ones((16,), jnp.int32)
        for k in range(b_per_w // 16):
            rows16 = lanes + (k * 16)
            idxw_v[pl.ds(k * 16, 16)] = plsc.load_gather(idx_v, [rows16, col0])
            idxl_v[pl.ds(k * 16, 16)] = plsc.load_gather(idx_v, [rows16, col1])
        copies = []
        for j in range(chunks):
            sl = pl.ds(j * _IDXW, _IDXW)
            copies.append(
                pltpu.async_copy(table_hbm.at[idxw_v.at[sl]], roww_v.at[sl], sem)
            )
            copies.append(
                pltpu.async_copy(table_hbm.at[idxl_v.at[sl]], rowl_v.at[sl], sem)
            )
        for c in copies:
            c.wait()
        i_blk = wid // (_BLK // b_per_w)
        k0 = (base % _BLK) // _SUB
        for t in range(b_per_w // _SUB):
            rsl = pl.ds(t * _SUB, _SUB)
            dst_r = pl.ds(_SUB * i_blk, _SUB)
            pltpu.sync_copy(
                roww_v.at[rsl],
                out_hbm.at[0, dst_r, pl.ds(dim * (k0 + t), dim)],
            )
            pltpu.sync_copy(
                rowl_v.at[rsl],
                out_hbm.at[1, dst_r, pl.ds(dim * (k0 + t), dim)],
            )

    return gather_k(table, x)


def _mlp_body(w_ref, l_ref, w1w_ref, w1l_ref, b1_ref, w2_ref, b2_ref, out_ref):
    z = jnp.dot(w_ref[0], w1w_ref[...], preferred_element_type=jnp.float32)
    z = z + jnp.dot(l_ref[0], w1l_ref[...], preferred_element_type=jnp.float32)
    z = jnp.maximum(z + b1_ref[...], 0.0)
    y = jnp.dot(z, w2_ref[...], preferred_element_type=jnp.float32) + b2_ref[...]
    stages = out_ref.shape[1]
    pack = y.shape[1] // stages
    for k in range(pack):
        out_ref[pl.ds(_SUB * k, _SUB), :] = y[:, stages * k:stages * (k + 1)]


@functools.partial(jax.jit, static_argnums=(7,))
def _tc_mlp(hw, hl, W1wp, W1lp, b1p, W2p, b2p, stages):
    _, rows, width = hw.shape
    hidden = W1wp.shape[1]
    out_packed = W2p.shape[1]
    pack = out_packed // stages
    grid = (rows // _SUB,)
    return pl.pallas_call(
        _mlp_body,
        grid=grid,
        in_specs=[
            pl.BlockSpec((1, _SUB, width), lambda i: (0, i, 0)),
            pl.BlockSpec((1, _SUB, width), lambda i: (1, i, 0)),
            pl.BlockSpec((width, hidden), lambda i: (0, 0)),
            pl.BlockSpec((width, hidden), lambda i: (0, 0)),
            pl.BlockSpec((1, hidden), lambda i: (0, 0)),
            pl.BlockSpec((hidden, out_packed), lambda i: (0, 0)),
            pl.BlockSpec((1, out_packed), lambda i: (0, 0)),
        ],
        out_specs=pl.BlockSpec((_SUB * pack, stages), lambda i: (i, 0)),
        out_shape=jax.ShapeDtypeStruct((rows * pack, stages), jnp.float32),
    )(hw, hl, W1wp, W1lp, b1p, W2p, b2p)


def kernel(x, emb, W1, b1, W2, b2):
    batch = x.shape[0]
    dim = emb.shape[1]
    stages = W2.shape[1]
    pack = 128 // dim
    gathered = _sc_gather(emb, x, batch, dim)
    eye = jnp.eye(pack, dtype=jnp.float32)
    W1wp = jnp.kron(eye, W1[:dim])
    W1lp = jnp.kron(eye, W1[dim:])
    b1p = jnp.tile(b1, pack).reshape(1, -1)
    W2p = jnp.kron(eye, W2)
    b2p = jnp.tile(b2, pack).reshape(1, -1)
    return _tc_mlp(gathered, gathered, W1wp, W1lp, b1p, W2p, b2p, stages)


# speedup vs baseline: 1.0362x; 1.0362x over previous
import functools

import jax
import jax.numpy as jnp
from jax import lax
from jax.experimental import pallas as pl
from jax.experimental.pallas import tpu as pltpu
from jax.experimental.pallas import tpu_sc as plsc

try:
    _INFO = plsc.get_sparse_core_info()
    _NC = _INFO.num_cores
    _NS = _INFO.num_subcores
except ValueError:
    _NC, _NS = 2, 16
_NW = _NC * _NS
_IDXW = 128
_BLK = 2048
_SUB = 256


@functools.partial(jax.jit, static_argnums=(2, 3))
def _sc_gather(table, x, batch, dim):
    b_per_w = batch // _NW
    chunks = b_per_w // _IDXW
    pack = 128 // dim
    mesh = plsc.VectorSubcoreMesh(core_axis_name="c", subcore_axis_name="s")

    @functools.partial(
        pl.kernel,
        mesh=mesh,
        compiler_params=pltpu.CompilerParams(
            use_tc_tiling_on_sc=False, needs_layout_passes=False
        ),
        out_type=jax.ShapeDtypeStruct((2, batch // pack, 128), jnp.float32),
        scratch_types=[
            pltpu.VMEM((b_per_w, 128), jnp.int32),
            pltpu.VMEM((b_per_w,), jnp.int32),
            pltpu.VMEM((b_per_w,), jnp.int32),
            pltpu.VMEM((b_per_w, dim), jnp.float32),
            pltpu.VMEM((b_per_w, dim), jnp.float32),
            pltpu.SemaphoreType.DMA,
        ],
    )
    def gather_k(table_hbm, x_hbm, out_hbm, idx_v, idxw_v, idxl_v,
                 roww_v, rowl_v, sem):
        wid = lax.axis_index("s") * _NC + lax.axis_index("c")
        base = wid * b_per_w
        pltpu.sync_copy(x_hbm.at[pl.ds(base, b_per_w)], idx_v)
        lanes = lax.iota(jnp.int32, 16)
        col0 = jnp.zeros((16,), jnp.int32)
        col1 = jnp.ones((16,), jnp.int32)
        for k in range(b_per_w // 16):
            rows16 = lanes + (k * 16)
            idxw_v[pl.ds(k * 16, 16)] = plsc.load_gather(idx_v, [rows16, col0])
            idxl_v[pl.ds(k * 16, 16)] = plsc.load_gather(idx_v, [rows16, col1])
        copies = []
        for j in range(chunks):
            sl = pl.ds(j * _IDXW, _IDXW)
            copies.append(
                pltpu.async_copy(table_hbm.at[idxw_v.at[sl]], roww_v.at[sl], sem)
            )
            copies.append(
                pltpu.async_copy(table_hbm.at[idxl_v.at[sl]], rowl_v.at[sl], sem)
            )
        for c in copies:
            c.wait()
        i_blk = wid // (_BLK // b_per_w)
        k0 = (base % _BLK) // _SUB
        for t in range(b_per_w // _SUB):
            rsl = pl.ds(t * _SUB, _SUB)
            dst_r = pl.ds(_SUB * i_blk, _SUB)
            pltpu.sync_copy(
                roww_v.at[rsl],
                out_hbm.at[0, dst_r, pl.ds(dim * (k0 + t), dim)],
            )
            pltpu.sync_copy(
                rowl_v.at[rsl],
                out_hbm.at[1, dst_r, pl.ds(dim * (k0 + t), dim)],
            )

    return gather_k(table, x)


def _mlp_body(w_ref, l_ref, w1w_ref, w1l_ref, b1_ref, w2_ref, b2_ref, out_ref):
    z = jnp.dot(w_ref[0], w1w_ref[...], preferred_element_type=jnp.float32)
    z = z + jnp.dot(l_ref[0], w1l_ref[...], preferred_element_type=jnp.float32)
    z = jnp.maximum(z + b1_ref[...], 0.0)
    y = jnp.dot(z, w2_ref[...], preferred_element_type=jnp.float32) + b2_ref[...]
    stages = out_ref.shape[1]
    pack = y.shape[1] // stages
    for k in range(pack):
        out_ref[pl.ds(_SUB * k, _SUB), :] = y[:, stages * k:stages * (k + 1)]


@functools.partial(jax.jit, static_argnums=(7,))
def _tc_mlp(hw, hl, W1wp, W1lp, b1p, W2p, b2p, stages):
    _, rows, width = hw.shape
    hidden = W1wp.shape[1]
    out_packed = W2p.shape[1]
    pack = out_packed // stages
    grid = (rows // _SUB,)
    return pl.pallas_call(
        _mlp_body,
        grid=grid,
        in_specs=[
            pl.BlockSpec((1, _SUB, width), lambda i: (0, i, 0)),
            pl.BlockSpec((1, _SUB, width), lambda i: (1, i, 0)),
            pl.BlockSpec((width, hidden), lambda i: (0, 0)),
            pl.BlockSpec((width, hidden), lambda i: (0, 0)),
            pl.BlockSpec((1, hidden), lambda i: (0, 0)),
            pl.BlockSpec((hidden, out_packed), lambda i: (0, 0)),
            pl.BlockSpec((1, out_packed), lambda i: (0, 0)),
        ],
        out_specs=pl.BlockSpec((_SUB * pack, stages), lambda i: (i, 0)),
        out_shape=jax.ShapeDtypeStruct((rows * pack, stages), jnp.float32),
    )(hw, hl, W1wp, W1lp, b1p, W2p, b2p)


def kernel(x, emb, W1, b1, W2, b2):
    batch = x.shape[0]
    dim = emb.shape[1]
    stages = W2.shape[1]
    pack = 128 // dim
    x_pad = jnp.pad(x, ((0, 0), (0, 128 - x.shape[1])))
    gathered = _sc_gather(emb, x_pad, batch, dim)
    eye = jnp.eye(pack, dtype=jnp.float32)
    W1wp = jnp.kron(eye, W1[:dim])
    W1lp = jnp.kron(eye, W1[dim:])
    b1p = jnp.tile(b1, pack).reshape(1, -1)
    W2p = jnp.kron(eye, W2)
    b2p = jnp.tile(b2, pack).reshape(1, -1)
    return _tc_mlp(gathered, gathered, W1wp, W1lp, b1p, W2p, b2p, stages)


# speedup vs baseline: 1.1694x; 1.1286x over previous
import functools

import jax
import jax.numpy as jnp
from jax import lax
from jax.experimental import pallas as pl
from jax.experimental.pallas import tpu as pltpu
from jax.experimental.pallas import tpu_sc as plsc

try:
    _INFO = plsc.get_sparse_core_info()
    _NC = _INFO.num_cores
    _NS = _INFO.num_subcores
except ValueError:
    _NC, _NS = 2, 16
_NW = _NC * _NS
_IDXW = 128
_BLK = 2048
_SUB = 256


@functools.partial(jax.jit, static_argnums=(2, 3))
def _sc_gather(table, x, batch, dim):
    b_per_w = batch // _NW
    chunks = b_per_w // _IDXW
    pack = 128 // dim
    mesh = plsc.VectorSubcoreMesh(core_axis_name="c", subcore_axis_name="s")

    @functools.partial(
        pl.kernel,
        mesh=mesh,
        compiler_params=pltpu.CompilerParams(
            use_tc_tiling_on_sc=False, needs_layout_passes=False
        ),
        out_type=jax.ShapeDtypeStruct((2, batch // pack, 128), jnp.float32),
        scratch_types=[
            pltpu.VMEM((b_per_w, 128), jnp.int32),
            pltpu.VMEM((b_per_w,), jnp.int32),
            pltpu.VMEM((b_per_w,), jnp.int32),
            pltpu.VMEM((b_per_w, dim), jnp.float32),
            pltpu.VMEM((b_per_w, dim), jnp.float32),
            pltpu.SemaphoreType.DMA,
        ],
    )
    def gather_k(table_hbm, x_hbm, out_hbm, idx_v, idxw_v, idxl_v,
                 roww_v, rowl_v, sem):
        wid = lax.axis_index("s") * _NC + lax.axis_index("c")
        base = wid * b_per_w
        pltpu.sync_copy(x_hbm.at[pl.ds(base, b_per_w)], idx_v)
        lanes = lax.iota(jnp.int32, 16)
        col0 = jnp.zeros((16,), jnp.int32)
        col1 = jnp.ones((16,), jnp.int32)
        for k in range(b_per_w // 16):
            rows16 = lanes + (k * 16)
            idxw_v[pl.ds(k * 16, 16)] = plsc.load_gather(idx_v, [rows16, col0])
            idxl_v[pl.ds(k * 16, 16)] = plsc.load_gather(idx_v, [rows16, col1])
        copies = []
        for j in range(chunks):
            sl = pl.ds(j * _IDXW, _IDXW)
            copies.append(
                pltpu.async_copy(table_hbm.at[idxw_v.at[sl]], roww_v.at[sl], sem)
            )
            copies.append(
                pltpu.async_copy(table_hbm.at[idxl_v.at[sl]], rowl_v.at[sl], sem)
            )
        for c in copies:
            c.wait()
        i_blk = wid // (_BLK // b_per_w)
        k0 = (base % _BLK) // _SUB
        for t in range(b_per_w // _SUB):
            rsl = pl.ds(t * _SUB, _SUB)
            dst_r = pl.ds(_SUB * i_blk, _SUB)
            pltpu.sync_copy(
                roww_v.at[rsl],
                out_hbm.at[0, dst_r, pl.ds(dim * (k0 + t), dim)],
            )
            pltpu.sync_copy(
                rowl_v.at[rsl],
                out_hbm.at[1, dst_r, pl.ds(dim * (k0 + t), dim)],
            )

    return gather_k(table, x)


def _repack_body(xt_ref, out_ref):
    x = xt_ref[...]
    dim, cols = x.shape
    pack = 128 // dim
    x3 = x.T.reshape(cols // pack, pack, dim)
    for j in range(pack):
        out_ref[:, dim * j:dim * (j + 1)] = x3[:, j, :]


@jax.jit
def _tc_repack(embT):
    dim, rows = embT.shape
    pack = 128 // dim
    block_c = 12800
    grid = (pl.cdiv(rows, block_c),)
    out_rows = grid[0] * block_c // pack
    return pl.pallas_call(
        _repack_body,
        grid=grid,
        in_specs=[pl.BlockSpec((dim, block_c), lambda i: (0, i))],
        out_specs=pl.BlockSpec((block_c // pack, 128), lambda i: (i, 0)),
        out_shape=jax.ShapeDtypeStruct((out_rows, 128), jnp.float32),
    )(embT)


def _mlp_body(w_ref, l_ref, w1w_ref, w1l_ref, b1_ref, w2_ref, b2_ref, out_ref):
    z = jnp.dot(w_ref[0], w1w_ref[...], preferred_element_type=jnp.float32)
    z = z + jnp.dot(l_ref[0], w1l_ref[...], preferred_element_type=jnp.float32)
    z = jnp.maximum(z + b1_ref[...], 0.0)
    y = jnp.dot(z, w2_ref[...], preferred_element_type=jnp.float32) + b2_ref[...]
    stages = out_ref.shape[1]
    pack = y.shape[1] // stages
    for k in range(pack):
        out_ref[pl.ds(_SUB * k, _SUB), :] = y[:, stages * k:stages * (k + 1)]


@functools.partial(jax.jit, static_argnums=(7,))
def _tc_mlp(hw, hl, W1wp, W1lp, b1p, W2p, b2p, stages):
    _, rows, width = hw.shape
    hidden = W1wp.shape[1]
    out_packed = W2p.shape[1]
    pack = out_packed // stages
    grid = (rows // _SUB,)
    return pl.pallas_call(
        _mlp_body,
        grid=grid,
        in_specs=[
            pl.BlockSpec((1, _SUB, width), lambda i: (0, i, 0)),
            pl.BlockSpec((1, _SUB, width), lambda i: (1, i, 0)),
            pl.BlockSpec((width, hidden), lambda i: (0, 0)),
            pl.BlockSpec((width, hidden), lambda i: (0, 0)),
            pl.BlockSpec((1, hidden), lambda i: (0, 0)),
            pl.BlockSpec((hidden, out_packed), lambda i: (0, 0)),
            pl.BlockSpec((1, out_packed), lambda i: (0, 0)),
        ],
        out_specs=pl.BlockSpec((_SUB * pack, stages), lambda i: (i, 0)),
        out_shape=jax.ShapeDtypeStruct((rows * pack, stages), jnp.float32),
    )(hw, hl, W1wp, W1lp, b1p, W2p, b2p)


def kernel(x, emb, W1, b1, W2, b2):
    batch = x.shape[0]
    dim = emb.shape[1]
    stages = W2.shape[1]
    pack = 128 // dim
    x_pad = jnp.pad(x, ((0, 0), (0, 128 - x.shape[1])))
    packed = _tc_repack(emb.T)
    table_lin = packed.reshape(packed.shape[0] * (128 // dim), dim)
    gathered = _sc_gather(table_lin, x_pad, batch, dim)
    eye = jnp.eye(pack, dtype=jnp.float32)
    W1wp = jnp.kron(eye, W1[:dim])
    W1lp = jnp.kron(eye, W1[dim:])
    b1p = jnp.tile(b1, pack).reshape(1, -1)
    W2p = jnp.kron(eye, W2)
    b2p = jnp.tile(b2, pack).reshape(1, -1)
    return _tc_mlp(gathered, gathered, W1wp, W1lp, b1p, W2p, b2p, stages)


# speedup vs baseline: 1.4390x; 1.2305x over previous
import functools

import jax
import jax.numpy as jnp
from jax import lax
from jax.experimental import pallas as pl
from jax.experimental.pallas import tpu as pltpu
from jax.experimental.pallas import tpu_sc as plsc

try:
    _INFO = plsc.get_sparse_core_info()
    _NC = _INFO.num_cores
    _NS = _INFO.num_subcores
except ValueError:
    _NC, _NS = 2, 16
_NW = _NC * _NS
_IDXW = 128
_BLK = 2048
_SUB = 256


@functools.partial(jax.jit, static_argnums=(2, 3))
def _sc_gather(table, x, batch, dim):
    b_per_w = batch // _NW
    chunks = b_per_w // _IDXW
    pack = 128 // dim
    mesh = plsc.VectorSubcoreMesh(core_axis_name="c", subcore_axis_name="s")

    @functools.partial(
        pl.kernel,
        mesh=mesh,
        compiler_params=pltpu.CompilerParams(
            use_tc_tiling_on_sc=False, needs_layout_passes=False
        ),
        out_type=jax.ShapeDtypeStruct((2, batch // pack, 128), jnp.float32),
        scratch_types=[
            pltpu.VMEM((2 * chunks, _IDXW), jnp.int32),
            pltpu.VMEM((b_per_w, dim), jnp.float32),
            pltpu.VMEM((b_per_w, dim), jnp.float32),
            pltpu.SemaphoreType.DMA,
        ],
    )
    def gather_k(table_hbm, x_hbm, out_hbm, idx_v, roww_v, rowl_v, sem):
        wid = lax.axis_index("s") * _NC + lax.axis_index("c")
        base = wid * b_per_w
        pltpu.sync_copy(x_hbm.at[pl.ds(2 * chunks * wid, 2 * chunks)], idx_v)
        copies = []
        for j in range(2 * chunks):
            dst = roww_v if j % 2 == 0 else rowl_v
            copies.append(
                pltpu.async_copy(
                    table_hbm.at[idx_v.at[j]],
                    dst.at[pl.ds((j // 2) * _IDXW, _IDXW)],
                    sem,
                )
            )
        for c in copies:
            c.wait()
        i_blk = wid // (_BLK // b_per_w)
        k0 = (base % _BLK) // _SUB
        for t in range(b_per_w // _SUB):
            rsl = pl.ds(t * _SUB, _SUB)
            dst_r = pl.ds(_SUB * i_blk, _SUB)
            pltpu.sync_copy(
                roww_v.at[rsl],
                out_hbm.at[0, dst_r, pl.ds(dim * (k0 + t), dim)],
            )
            pltpu.sync_copy(
                rowl_v.at[rsl],
                out_hbm.at[1, dst_r, pl.ds(dim * (k0 + t), dim)],
            )

    return gather_k(table, x)


def _repack_body(xt_ref, out_ref):
    x = xt_ref[...]
    dim, cols = x.shape
    pack = 128 // dim
    x3 = x.T.reshape(cols // pack, pack, dim)
    for j in range(pack):
        out_ref[:, dim * j:dim * (j + 1)] = x3[:, j, :]


@jax.jit
def _tc_repack(embT):
    dim, rows = embT.shape
    pack = 128 // dim
    block_c = 12800
    grid = (pl.cdiv(rows, block_c),)
    out_rows = grid[0] * block_c // pack
    return pl.pallas_call(
        _repack_body,
        grid=grid,
        in_specs=[pl.BlockSpec((dim, block_c), lambda i: (0, i))],
        out_specs=pl.BlockSpec((block_c // pack, 128), lambda i: (i, 0)),
        out_shape=jax.ShapeDtypeStruct((out_rows, 128), jnp.float32),
    )(embT)


def _mlp_body(w_ref, l_ref, w1w_ref, w1l_ref, b1_ref, w2_ref, b2_ref, out_ref):
    z = jnp.dot(w_ref[0], w1w_ref[...], preferred_element_type=jnp.float32)
    z = z + jnp.dot(l_ref[0], w1l_ref[...], preferred_element_type=jnp.float32)
    z = jnp.maximum(z + b1_ref[...], 0.0)
    y = jnp.dot(z, w2_ref[...], preferred_element_type=jnp.float32) + b2_ref[...]
    stages = out_ref.shape[1]
    pack = y.shape[1] // stages
    for k in range(pack):
        out_ref[pl.ds(_SUB * k, _SUB), :] = y[:, stages * k:stages * (k + 1)]


@functools.partial(jax.jit, static_argnums=(7,))
def _tc_mlp(hw, hl, W1wp, W1lp, b1p, W2p, b2p, stages):
    _, rows, width = hw.shape
    hidden = W1wp.shape[1]
    out_packed = W2p.shape[1]
    pack = out_packed // stages
    grid = (rows // _SUB,)
    return pl.pallas_call(
        _mlp_body,
        grid=grid,
        in_specs=[
            pl.BlockSpec((1, _SUB, width), lambda i: (0, i, 0)),
            pl.BlockSpec((1, _SUB, width), lambda i: (1, i, 0)),
            pl.BlockSpec((width, hidden), lambda i: (0, 0)),
            pl.BlockSpec((width, hidden), lambda i: (0, 0)),
            pl.BlockSpec((1, hidden), lambda i: (0, 0)),
            pl.BlockSpec((hidden, out_packed), lambda i: (0, 0)),
            pl.BlockSpec((1, out_packed), lambda i: (0, 0)),
        ],
        out_specs=pl.BlockSpec((_SUB * pack, stages), lambda i: (i, 0)),
        out_shape=jax.ShapeDtypeStruct((rows * pack, stages), jnp.float32),
    )(hw, hl, W1wp, W1lp, b1p, W2p, b2p)


def kernel(x, emb, W1, b1, W2, b2):
    batch = x.shape[0]
    dim = emb.shape[1]
    stages = W2.shape[1]
    pack = 128 // dim
    x_chunks = x.reshape(batch // 128, 128, 2).transpose(0, 2, 1).reshape(
        batch // 64, 128)
    packed = _tc_repack(emb.T)
    table_lin = packed.reshape(packed.shape[0] * (128 // dim), dim)
    gathered = _sc_gather(table_lin, x_chunks, batch, dim)
    eye = jnp.eye(pack, dtype=jnp.float32)
    W1wp = jnp.kron(eye, W1[:dim])
    W1lp = jnp.kron(eye, W1[dim:])
    b1p = jnp.tile(b1, pack).reshape(1, -1)
    W2p = jnp.kron(eye, W2)
    b2p = jnp.tile(b2, pack).reshape(1, -1)
    return _tc_mlp(gathered, gathered, W1wp, W1lp, b1p, W2p, b2p, stages)


# speedup vs baseline: 1.6414x; 1.1407x over previous
import functools

import jax
import jax.numpy as jnp
from jax import lax
from jax.experimental import pallas as pl
from jax.experimental.pallas import tpu as pltpu
from jax.experimental.pallas import tpu_sc as plsc

try:
    _INFO = plsc.get_sparse_core_info()
    _NC = _INFO.num_cores
    _NS = _INFO.num_subcores
except ValueError:
    _NC, _NS = 2, 16
_NW = _NC * _NS
_IDXW = 128
_BLK = 2048
_SUB = 256


@functools.partial(jax.jit, static_argnums=(2, 3))
def _sc_gather(table, x, batch, dim):
    b_per_w = batch // _NW
    chunks = b_per_w // _IDXW
    pack = 128 // dim
    mesh = plsc.VectorSubcoreMesh(core_axis_name="c", subcore_axis_name="s")

    @functools.partial(
        pl.kernel,
        mesh=mesh,
        compiler_params=pltpu.CompilerParams(
            use_tc_tiling_on_sc=False, needs_layout_passes=False
        ),
        out_type=jax.ShapeDtypeStruct((2, batch // pack, 128), jnp.float32),
        scratch_types=[
            pltpu.VMEM((2 * chunks, _IDXW), jnp.int32),
            pltpu.VMEM((b_per_w, dim), jnp.float32),
            pltpu.VMEM((b_per_w, dim), jnp.float32),
            pltpu.SemaphoreType.DMA,
        ],
    )
    def gather_k(table_hbm, x_hbm, out_hbm, idx_v, roww_v, rowl_v, sem):
        wid = lax.axis_index("s") * _NC + lax.axis_index("c")
        base = wid * b_per_w
        pltpu.sync_copy(x_hbm.at[pl.ds(2 * chunks * wid, 2 * chunks)], idx_v)
        copies = []
        for j in range(2 * chunks):
            dst = roww_v if j % 2 == 0 else rowl_v
            copies.append(
                pltpu.async_copy(
                    table_hbm.at[idx_v.at[j]],
                    dst.at[pl.ds((j // 2) * _IDXW, _IDXW)],
                    sem,
                )
            )
        for c in copies:
            c.wait()
        i_blk = wid // (_BLK // b_per_w)
        k0 = (base % _BLK) // _SUB
        for t in range(b_per_w // _SUB):
            rsl = pl.ds(t * _SUB, _SUB)
            dst_r = pl.ds(_SUB * i_blk, _SUB)
            pltpu.sync_copy(
                roww_v.at[rsl],
                out_hbm.at[0, dst_r, pl.ds(dim * (k0 + t), dim)],
            )
            pltpu.sync_copy(
                rowl_v.at[rsl],
                out_hbm.at[1, dst_r, pl.ds(dim * (k0 + t), dim)],
            )

    return gather_k(table, x)


def _repack_body(xt_ref, out_ref):
    x = xt_ref[...]
    dim, cols = x.shape
    pack = 128 // dim
    x3 = x.T.reshape(cols // pack, pack, dim)
    for j in range(pack):
        out_ref[:, dim * j:dim * (j + 1)] = x3[:, j, :]


@jax.jit
def _tc_repack(embT):
    dim, rows = embT.shape
    pack = 128 // dim
    block_c = 12800
    grid = (pl.cdiv(rows, block_c),)
    out_rows = grid[0] * block_c // pack
    return pl.pallas_call(
        _repack_body,
        grid=grid,
        in_specs=[pl.BlockSpec((dim, block_c), lambda i: (0, i))],
        out_specs=pl.BlockSpec((block_c // pack, 128), lambda i: (i, 0)),
        out_shape=jax.ShapeDtypeStruct((out_rows, 128), jnp.float32),
    )(embT)


def _mlp_body(w_ref, l_ref, w1w_ref, w1l_ref, b1_ref, w2_ref, b2_ref, out_ref):
    w = w_ref[0]
    l = l_ref[0]
    dn = (((0,), (1,)), ((), ()))
    z = lax.dot_general(w1w_ref[...], w, dn, preferred_element_type=jnp.float32)
    z = z + lax.dot_general(w1l_ref[...], l, dn,
                            preferred_element_type=jnp.float32)
    z = jnp.maximum(z + b1_ref[...], 0.0)
    dn2 = (((0,), (0,)), ((), ()))
    y = lax.dot_general(w2_ref[...], z, dn2,
                        preferred_element_type=jnp.float32) + b2_ref[...]
    stages = out_ref.shape[0]
    pack = y.shape[0] // stages
    y3 = y.reshape(pack, stages, _SUB)
    for k in range(pack):
        out_ref[:, pl.ds(_SUB * k, _SUB)] = y3[k]


@functools.partial(jax.jit, static_argnums=(7,))
def _tc_mlp(hw, hl, W1wp, W1lp, b1p, W2p, b2p, stages):
    _, rows, width = hw.shape
    hidden = W1wp.shape[1]
    out_packed = W2p.shape[1]
    pack = out_packed // stages
    grid = (rows // _SUB,)
    return pl.pallas_call(
        _mlp_body,
        grid=grid,
        in_specs=[
            pl.BlockSpec((1, _SUB, width), lambda i: (0, i, 0)),
            pl.BlockSpec((1, _SUB, width), lambda i: (1, i, 0)),
            pl.BlockSpec((width, hidden), lambda i: (0, 0)),
            pl.BlockSpec((width, hidden), lambda i: (0, 0)),
            pl.BlockSpec((hidden, 1), lambda i: (0, 0)),
            pl.BlockSpec((hidden, out_packed), lambda i: (0, 0)),
            pl.BlockSpec((out_packed, 1), lambda i: (0, 0)),
        ],
        out_specs=pl.BlockSpec((stages, _SUB * pack), lambda i: (0, i)),
        out_shape=jax.ShapeDtypeStruct((stages, rows * pack), jnp.float32),
    )(hw, hl, W1wp, W1lp, b1p, W2p, b2p)


def kernel(x, emb, W1, b1, W2, b2):
    batch = x.shape[0]
    dim = emb.shape[1]
    stages = W2.shape[1]
    pack = 128 // dim
    x_chunks = x.reshape(batch // 128, 128, 2).transpose(0, 2, 1).reshape(
        batch // 64, 128)
    packed = _tc_repack(emb.T)
    table_lin = packed.reshape(packed.shape[0] * (128 // dim), dim)
    gathered = _sc_gather(table_lin, x_chunks, batch, dim)
    eye = jnp.eye(pack, dtype=jnp.float32)
    W1wp = jnp.kron(eye, W1[:dim])
    W1lp = jnp.kron(eye, W1[dim:])
    b1p = jnp.tile(b1, pack).reshape(-1, 1)
    W2p = jnp.kron(eye, W2)
    b2p = jnp.tile(b2, pack).reshape(-1, 1)
    out_t = _tc_mlp(gathered, gathered, W1wp, W1lp, b1p, W2p, b2p, stages)
    return out_t.T
